# baseline (speedup 1.0000x reference)
import jax
import jax.numpy as jnp
from jax import lax
from jax.experimental import pallas as pl
from jax.experimental.pallas import tpu as pltpu

B = 4
N = 256
NODES = B * N
D = 512
EMBED = 56
THRESH_SQ = 0.3 * 0.3
LAYERS = 9


def _leaky(v):
    return jnp.where(v >= 0, v, 0.2 * v)


def _lane_bcast(col, ones_col):
    return lax.dot_general(ones_col, col, (((1,), (1,)), ((), ())),
                           preferred_element_type=jnp.float32)


def _gnn_kernel(x_ref, am_ref, el_ref, po_ref, xyzw_ref, xyzb_ref, gatw_ref,
                asrc_ref, adst_ref, gatb_ref, ew_ref, eb_ref, out_ref,
                h_ref, minf_ref):
    f32 = jnp.float32
    step = pl.program_id(0)
    ones_col = jnp.ones((N, 1), f32)

    @pl.when(step == 0)
    def _init():
        xv = x_ref[...]
        ridx = xv[:, 0:1].astype(jnp.int32)
        aidx = xv[:, 1:2].astype(jnp.int32)
        cidx = xv[:, 2:3].astype(jnp.int32)

        etab = jnp.concatenate([
            jnp.pad(am_ref[...], ((0, 0), (0, D - EMBED))),
            jnp.pad(el_ref[...], ((0, 0), (EMBED, D - 2 * EMBED))),
            jnp.pad(po_ref[...], ((0, 2), (2 * EMBED, D - 3 * EMBED)))],
            axis=0)
        col = lax.broadcasted_iota(jnp.int32, (NODES, 48), 1)
        tgt = jnp.where(col < 20, ridx,
                        jnp.where(col < 25, aidx + 20, cidx + 25))
        oh = jnp.where(tgt == col, f32(1.0), f32(0.0))
        emb = jnp.dot(oh, etab, preferred_element_type=f32)

        xyzw = jnp.pad(xyzw_ref[...], ((0, 0), (3 * EMBED, 0)))
        xyzb = jnp.pad(xyzb_ref[...], ((0, 0), (3 * EMBED, 0)))
        xyz = jnp.maximum(
            jnp.dot(xv[:, 3:6], xyzw, preferred_element_type=f32) + xyzb,
            0.0)
        h0 = emb + xyz
        h_ref[...] = h0

        lane = lax.broadcasted_iota(jnp.int32, (N, D), 1)
        rowj = lax.broadcasted_iota(jnp.int32, (N, N), 0)
        coli = lax.broadcasted_iota(jnp.int32, (N, N), 1)
        for b in range(B):
            hb = h0[b * N:(b + 1) * N, :]
            p = jnp.where(lane >= 3, hb, 0.0)
            q = p - jnp.mean(p, axis=0, keepdims=True)
            nrm = jnp.sum(q * q, axis=1, keepdims=True)
            gram = lax.dot_general(q, q, (((1,), (1,)), ((), ())),
                                   preferred_element_type=f32)
            d2 = nrm + _lane_bcast(nrm, ones_col) - 2.0 * gram
            valid = ((d2 < THRESH_SQ) & (coli < rowj)) | (coli == rowj)
            minf_ref[b] = jnp.where(valid, f32(0.0), -jnp.inf)

    h = h_ref[...]
    z = jnp.dot(h, gatw_ref[0], preferred_element_type=f32)
    aa = jnp.concatenate([asrc_ref[0], adst_ref[0]], axis=0)
    zsd = lax.dot_general(z, aa, (((1,), (1,)), ((), ())),
                          preferred_element_type=f32)
    bias = gatb_ref[0]
    new_h = []
    for b in range(B):
        sl = slice(b * N, (b + 1) * N)
        alpha = _leaky(zsd[sl, 1:2] + _lane_bcast(zsd[sl, 0:1], ones_col))
        am = alpha + minf_ref[b]
        mx = jnp.max(am, axis=1, keepdims=True)
        e = jnp.exp(am - mx)
        coef = e / jnp.sum(e, axis=1, keepdims=True)
        ob = jnp.dot(coef, z[sl, :], preferred_element_type=f32)
        new_h.append(jnp.maximum(ob + bias + h[sl, :], 0.0))
    hn = jnp.concatenate(new_h, axis=0)
    h_ref[...] = hn

    @pl.when(step == LAYERS - 1)
    def _fin():
        en = jnp.dot(hn, ew_ref[...], preferred_element_type=f32)
        selr = lax.broadcasted_iota(jnp.int32, (B, NODES), 0)
        selc = lax.broadcasted_iota(jnp.int32, (B, NODES), 1)
        sel = jnp.where(selc // N == selr, f32(1.0 / N), f32(0.0))
        out_ref[...] = (jnp.dot(sel, en, preferred_element_type=f32)
                        + eb_ref[...])


def kernel(x, amino_embed, element_embed, position_embed, xyz_W, xyz_b,
           gat_W, gat_att_src, gat_att_dst, gat_b, energy_W, energy_b):
    f32 = jnp.float32
    const = lambda shape: pl.BlockSpec(shape, lambda l: (0,) * len(shape))
    out = pl.pallas_call(
        _gnn_kernel,
        grid=(LAYERS,),
        in_specs=[
            const((NODES, 6)),
            const((20, EMBED)), const((5, EMBED)), const((21, EMBED)),
            const((3, 344)), const((1, 344)),
            pl.BlockSpec((1, D, D), lambda l: (l, 0, 0)),
            pl.BlockSpec((1, 1, D), lambda l: (l, 0, 0)),
            pl.BlockSpec((1, 1, D), lambda l: (l, 0, 0)),
            pl.BlockSpec((1, 1, D), lambda l: (l, 0, 0)),
            const((D, 1)), const((1, 1)),
        ],
        out_specs=const((B, 1)),
        scratch_shapes=[pltpu.VMEM((NODES, D), f32),
                        pltpu.VMEM((B, N, N), f32)],
        out_shape=jax.ShapeDtypeStruct((B, 1), f32),
    )(x.reshape(NODES, 6), amino_embed, element_embed, position_embed,
      xyz_W, xyz_b.reshape(1, -1), gat_W,
      gat_att_src.reshape(LAYERS, 1, D), gat_att_dst.reshape(LAYERS, 1, D),
      gat_b.reshape(LAYERS, 1, D), energy_W, energy_b.reshape(1, 1))
    return out
